# baseline (speedup 1.0000x reference)
import functools

import numpy as np
import jax
import jax.numpy as jnp
from jax.experimental import pallas as pl

DIM = 20
BBLK = 256


def _dot(a, b):
    return jax.lax.dot_general(
        a, b, (((1,), (0,)), ((), ())),
        precision=jax.lax.Precision.HIGHEST,
        preferred_element_type=jnp.float32)


def _fwd_kernel(x_ref, g1w_ref, b1_ref, w2_ref, b2_ref, w3_ref, b3_ref,
                w4_ref, b4_ref, f1_ref, g1_ref, f2_ref, g2_ref):
    x = x_ref[...]
    h1 = jnp.maximum(_dot(x, g1w_ref[...]) + b1_ref[...], 0.0)
    w2 = w2_ref[...]
    w3 = w3_ref[...]
    b2 = b2_ref[...]
    b3 = b3_ref[...]
    h3_parts = []
    for kt in range(5):
        h1k = h1[:, 128 * kt:128 * kt + 128]
        h2k = jnp.maximum(_dot(h1k, w2) + b2, 0.0)
        h3k = jnp.maximum(_dot(h2k, w3) + b3, 0.0)
        h3_parts.append(h3k)
    h3 = jnp.concatenate(h3_parts, axis=1)
    o = _dot(h3, w4_ref[...]) + b4_ref[...]

    f1_ref[...] = o[:, 0:20]
    f2_ref[...] = o[:, 60:80]

    ii = jax.lax.broadcasted_iota(jnp.int32, (1, DIM, DIM), 1)
    jj = jax.lax.broadcasted_iota(jnp.int32, (1, DIM, DIM), 2)
    d = (jj - ii + DIM) % DIM

    ga_sub = o[:, 20:40].reshape(x.shape[0], DIM, 1)
    ga_diag = o[:, 40:60].reshape(x.shape[0], DIM, 1)
    g1_ref[...] = (jnp.where(d == 0, ga_diag, 0.0)
                   + jnp.where(d == 19, ga_sub, 0.0))

    gb_sub = o[:, 80:100].reshape(x.shape[0], DIM, 1)
    gb_diag = o[:, 100:120].reshape(x.shape[0], DIM, 1)
    gb_sup = o[:, 120:140].reshape(x.shape[0], DIM, 1)
    g2_ref[...] = (jnp.where(d == 0, gb_diag, 0.0)
                   + jnp.where(d == 19, gb_sub, 0.0)
                   + jnp.where(d == 1, gb_sup, 0.0))


def _build_weights(Wa0, ba0, Wa1, ba1, Wa2, ba2, Wa3, ba3,
                   Wb0, bb0, Wb1, bb1, Wb2, bb2, Wb3, bb3):
    f32 = jnp.float32
    W0c = jnp.zeros((3, 32), f32)
    W0c = W0c.at[:, 0:16].set(Wa0)
    W0c = W0c.at[1:3, 16:32].set(Wb0)
    b0c = jnp.concatenate([ba0, bb0])

    M = np.zeros((3, DIM, DIM), np.float32)
    for t in range(3):
        for i in range(DIM):
            M[t, (i + t - 1) % DIM, i] = 1.0
    G1 = jnp.einsum('tji,tc->jic', jnp.asarray(M), W0c).reshape(DIM, DIM * 32)
    B1 = jnp.tile(b0c, DIM).reshape(1, DIM * 32)

    E4 = jnp.eye(4, dtype=f32)
    W1c = jnp.zeros((32, 64), f32)
    W1c = W1c.at[0:16, 0:32].set(Wa1)
    W1c = W1c.at[16:32, 32:64].set(Wb1)
    b1c = jnp.concatenate([ba1, bb1])
    W2chunk = jnp.einsum('ij,kc->ikjc', E4, W1c).reshape(128, 256)
    B2 = jnp.tile(b1c, 4).reshape(1, 256)

    W2c = jnp.zeros((64, 32), f32)
    W2c = W2c.at[0:32, 0:16].set(Wa2)
    W2c = W2c.at[32:64, 16:32].set(Wb2)
    b2c = jnp.concatenate([ba2, bb2])
    W3chunk = jnp.einsum('ij,kc->ikjc', E4, W2c).reshape(256, 128)
    B3 = jnp.tile(b2c, 4).reshape(1, 128)

    W3c = jnp.zeros((32, 7), f32)
    W3c = W3c.at[0:16, 0:3].set(Wa3)
    W3c = W3c.at[16:32, 3:7].set(Wb3)
    b3c = jnp.concatenate([ba3, bb3])
    E20 = jnp.eye(DIM, dtype=f32)
    W4 = jnp.einsum('rg,ij->irgj', W3c, E20).reshape(DIM * 32, 7 * DIM)
    B4 = jnp.repeat(b3c, DIM).reshape(1, 7 * DIM)
    return G1, B1, W2chunk, B2, W3chunk, B3, W4, B4


def kernel(x, Wa0, ba0, Wa1, ba1, Wa2, ba2, Wa3, ba3,
           Wb0, bb0, Wb1, bb1, Wb2, bb2, Wb3, bb3):
    batch = x.shape[0]
    G1, B1, W2chunk, B2, W3chunk, B3, W4, B4 = _build_weights(
        Wa0, ba0, Wa1, ba1, Wa2, ba2, Wa3, ba3,
        Wb0, bb0, Wb1, bb1, Wb2, bb2, Wb3, bb3)

    grid = (batch // BBLK,)
    full = lambda shape: pl.BlockSpec(shape, lambda b: (0,) * len(shape))
    f1, g1, f2, g2 = pl.pallas_call(
        _fwd_kernel,
        grid=grid,
        in_specs=[
            pl.BlockSpec((BBLK, DIM), lambda b: (b, 0)),
            full(G1.shape), full(B1.shape),
            full(W2chunk.shape), full(B2.shape),
            full(W3chunk.shape), full(B3.shape),
            full(W4.shape), full(B4.shape),
        ],
        out_specs=[
            pl.BlockSpec((BBLK, DIM), lambda b: (b, 0)),
            pl.BlockSpec((BBLK, DIM, DIM), lambda b: (b, 0, 0)),
            pl.BlockSpec((BBLK, DIM), lambda b: (b, 0)),
            pl.BlockSpec((BBLK, DIM, DIM), lambda b: (b, 0, 0)),
        ],
        out_shape=[
            jax.ShapeDtypeStruct((batch, DIM), jnp.float32),
            jax.ShapeDtypeStruct((batch, DIM, DIM), jnp.float32),
            jax.ShapeDtypeStruct((batch, DIM), jnp.float32),
            jax.ShapeDtypeStruct((batch, DIM, DIM), jnp.float32),
        ],
    )(x, G1, B1, W2chunk, B2, W3chunk, B3, W4, B4)
    return (f1.reshape(batch, DIM, 1), g1, f2.reshape(batch, DIM, 1), g2)


# speedup vs baseline: 1.6316x; 1.6316x over previous
import numpy as np
import jax
import jax.numpy as jnp
from jax.experimental import pallas as pl

DIM = 20
BBLK = 512


def _dot(a, b, precision=jax.lax.Precision.DEFAULT):
    return jax.lax.dot_general(
        a, b, (((1,), (0,)), ((), ())),
        precision=precision,
        preferred_element_type=jnp.float32)


def _fwd_kernel(x_ref, g1w_ref, b1_ref, w2_ref, b2_ref, w3_ref, b3_ref,
                w4_ref, b4_ref, f1_ref, g1_ref, f2_ref, g2_ref):
    x = x_ref[...]
    h1 = jnp.maximum(
        _dot(x, g1w_ref[...], jax.lax.Precision.HIGHEST) + b1_ref[...],
        0.0)
    w2 = w2_ref[...]
    w3 = w3_ref[...]
    b2 = b2_ref[...]
    b3 = b3_ref[...]
    h3_parts = []
    for kt in range(5):
        h1k = h1[:, 128 * kt:128 * kt + 128]
        h2k = jnp.maximum(_dot(h1k, w2) + b2, 0.0)
        h3k = jnp.maximum(_dot(h2k, w3) + b3, 0.0)
        h3_parts.append(h3k)
    h3 = jnp.concatenate(h3_parts, axis=1)
    o = _dot(h3, w4_ref[...]) + b4_ref[...]

    f1_ref[...] = o[:, 0:20]
    f2_ref[...] = o[:, 60:80]

    ii = jax.lax.broadcasted_iota(jnp.int32, (1, DIM, DIM), 1)
    jj = jax.lax.broadcasted_iota(jnp.int32, (1, DIM, DIM), 2)
    dd = (jj - ii + DIM) % DIM

    sa = o[:, 20:40][:, None, :]
    da = o[:, 40:60][:, None, :]
    g1_ref[...] = (jnp.where(dd == 0, da, 0.0)
                   + jnp.where(dd == DIM - 1, sa, 0.0))

    sb = o[:, 80:100][:, None, :]
    db = o[:, 100:120][:, None, :]
    pb = o[:, 120:140][:, None, :]
    g2_ref[...] = (jnp.where(dd == 0, db, 0.0)
                   + jnp.where(dd == DIM - 1, sb, 0.0)
                   + jnp.where(dd == 1, pb, 0.0))


def _build_weights(Wa0, ba0, Wa1, ba1, Wa2, ba2, Wa3, ba3,
                   Wb0, bb0, Wb1, bb1, Wb2, bb2, Wb3, bb3):
    f32 = jnp.float32
    W0c = jnp.zeros((3, 32), f32)
    W0c = W0c.at[:, 0:16].set(Wa0)
    W0c = W0c.at[1:3, 16:32].set(Wb0)
    b0c = jnp.concatenate([ba0, bb0])

    rr = np.array([[(i + t - 1) % DIM for i in range(DIM)] for t in range(3)])
    ic = np.array([[i for i in range(DIM)] for t in range(3)])
    G1 = jnp.zeros((DIM, DIM, 32), f32)
    G1 = G1.at[rr.ravel(), ic.ravel()].set(
        jnp.repeat(W0c, DIM, axis=0))
    G1 = G1.reshape(DIM, DIM * 32)
    B1 = jnp.tile(b0c, DIM).reshape(1, DIM * 32)

    W1c = jnp.zeros((32, 64), f32)
    W1c = W1c.at[0:16, 0:32].set(Wa1)
    W1c = W1c.at[16:32, 32:64].set(Wb1)
    b1c = jnp.concatenate([ba1, bb1])
    W2chunk = jnp.zeros((128, 256), f32)
    for k in range(4):
        W2chunk = W2chunk.at[32 * k:32 * k + 32, 64 * k:64 * k + 64].set(W1c)
    B2 = jnp.tile(b1c, 4).reshape(1, 256)

    W2c = jnp.zeros((64, 32), f32)
    W2c = W2c.at[0:32, 0:16].set(Wa2)
    W2c = W2c.at[32:64, 16:32].set(Wb2)
    b2c = jnp.concatenate([ba2, bb2])
    W3chunk = jnp.zeros((256, 128), f32)
    for k in range(4):
        W3chunk = W3chunk.at[64 * k:64 * k + 64, 32 * k:32 * k + 32].set(W2c)
    B3 = jnp.tile(b2c, 4).reshape(1, 128)

    W3c = jnp.zeros((32, 7), f32)
    W3c = W3c.at[0:16, 0:3].set(Wa3)
    W3c = W3c.at[16:32, 3:7].set(Wb3)
    b3c = jnp.concatenate([ba3, bb3])

    nodes = np.arange(DIM)
    ident = nodes
    sub_perm = (nodes - 1) % DIM
    sup_perm = (nodes + 1) % DIM
    group_cols = [ident, sub_perm, ident, ident, sub_perm, ident, sup_perm]

    W4 = jnp.zeros((DIM, 32, 7 * DIM), f32)
    for g, perm in enumerate(group_cols):
        cols = DIM * g + perm
        W4 = W4.at[nodes, :, cols].set(
            jnp.broadcast_to(W3c[:, g], (DIM, 32)))
    W4 = W4.reshape(DIM * 32, 7 * DIM)
    B4 = jnp.repeat(b3c, DIM).reshape(1, 7 * DIM)
    return G1, B1, W2chunk, B2, W3chunk, B3, W4, B4


def kernel(x, Wa0, ba0, Wa1, ba1, Wa2, ba2, Wa3, ba3,
           Wb0, bb0, Wb1, bb1, Wb2, bb2, Wb3, bb3):
    batch = x.shape[0]
    G1, B1, W2chunk, B2, W3chunk, B3, W4, B4 = _build_weights(
        Wa0, ba0, Wa1, ba1, Wa2, ba2, Wa3, ba3,
        Wb0, bb0, Wb1, bb1, Wb2, bb2, Wb3, bb3)

    grid = (batch // BBLK,)
    full = lambda shape: pl.BlockSpec(shape, lambda b: (0,) * len(shape))
    f1, g1, f2, g2 = pl.pallas_call(
        _fwd_kernel,
        grid=grid,
        in_specs=[
            pl.BlockSpec((BBLK, DIM), lambda b: (b, 0)),
            full(G1.shape), full(B1.shape),
            full(W2chunk.shape), full(B2.shape),
            full(W3chunk.shape), full(B3.shape),
            full(W4.shape), full(B4.shape),
        ],
        out_specs=[
            pl.BlockSpec((BBLK, DIM), lambda b: (b, 0)),
            pl.BlockSpec((BBLK, DIM, DIM), lambda b: (b, 0, 0)),
            pl.BlockSpec((BBLK, DIM), lambda b: (b, 0)),
            pl.BlockSpec((BBLK, DIM, DIM), lambda b: (b, 0, 0)),
        ],
        out_shape=[
            jax.ShapeDtypeStruct((batch, DIM), jnp.float32),
            jax.ShapeDtypeStruct((batch, DIM, DIM), jnp.float32),
            jax.ShapeDtypeStruct((batch, DIM), jnp.float32),
            jax.ShapeDtypeStruct((batch, DIM, DIM), jnp.float32),
        ],
    )(x, G1, B1, W2chunk, B2, W3chunk, B3, W4, B4)
    return (f1.reshape(batch, DIM, 1), g1, f2.reshape(batch, DIM, 1), g2)


# speedup vs baseline: 1.7132x; 1.0500x over previous
import numpy as np
import jax
import jax.numpy as jnp
from jax.experimental import pallas as pl

DIM = 20
BBLK = 512


def _dot(a, b, precision=jax.lax.Precision.DEFAULT):
    return jax.lax.dot_general(
        a, b, (((1,), (0,)), ((), ())),
        precision=precision,
        preferred_element_type=jnp.float32)


def _fwd_kernel(x_ref, g1w_ref, b1_ref, w2_ref, b2_ref, w3_ref, b3_ref,
                w4_ref, b4_ref, f1_ref, g1_ref, f2_ref, g2_ref):
    x = x_ref[...]
    h1 = jnp.maximum(
        _dot(x, g1w_ref[...], jax.lax.Precision.HIGHEST) + b1_ref[...],
        0.0)
    w2 = w2_ref[...]
    w3 = w3_ref[...]
    b2 = b2_ref[...]
    b3 = b3_ref[...]
    h3_parts = []
    for kt in range(5):
        h1k = h1[:, 128 * kt:128 * kt + 128]
        h2k = jnp.maximum(_dot(h1k, w2) + b2, 0.0)
        h3k = jnp.maximum(_dot(h2k, w3) + b3, 0.0)
        h3_parts.append(h3k)
    h3 = jnp.concatenate(h3_parts, axis=1)
    o = _dot(h3, w4_ref[...]) + b4_ref[...]

    f1_ref[...] = o[:, 0:20]
    f2_ref[...] = o[:, 60:80]

    ii = jax.lax.broadcasted_iota(jnp.int32, (1, DIM, DIM), 1)
    jj = jax.lax.broadcasted_iota(jnp.int32, (1, DIM, DIM), 2)
    dd = (jj - ii + DIM) % DIM

    sa = o[:, 20:40][:, None, :]
    da = o[:, 40:60][:, None, :]
    g1_ref[...] = jnp.where(dd == 0, da, jnp.where(dd == DIM - 1, sa, 0.0))

    sb = o[:, 80:100][:, None, :]
    db = o[:, 100:120][:, None, :]
    pb = o[:, 120:140][:, None, :]
    g2_ref[...] = jnp.where(
        dd == 0, db,
        jnp.where(dd == DIM - 1, sb, jnp.where(dd == 1, pb, 0.0)))


def _blockdiag2(A, B):
    za = jnp.zeros((A.shape[0], B.shape[1]), A.dtype)
    zb = jnp.zeros((B.shape[0], A.shape[1]), A.dtype)
    return jnp.concatenate(
        [jnp.concatenate([A, za], axis=1),
         jnp.concatenate([zb, B], axis=1)], axis=0)


def _build_weights(Wa0, ba0, Wa1, ba1, Wa2, ba2, Wa3, ba3,
                   Wb0, bb0, Wb1, bb1, Wb2, bb2, Wb3, bb3):
    f32 = jnp.float32
    W0c = jnp.concatenate(
        [Wa0, jnp.concatenate([jnp.zeros((1, 16), f32), Wb0], axis=0)],
        axis=1)
    b0c = jnp.concatenate([ba0, bb0])

    M = np.zeros((3, DIM, DIM, 1), np.float32)
    for t in range(3):
        for i in range(DIM):
            M[t, (i + t - 1) % DIM, i, 0] = 1.0
    G1 = (jnp.asarray(M) * W0c[:, None, None, :]).sum(0)
    G1 = G1.reshape(DIM, DIM * 32)
    B1 = jnp.tile(b0c, DIM).reshape(1, DIM * 32)

    E4 = np.eye(4, dtype=np.float32)[:, None, :, None]
    W1c = _blockdiag2(Wa1, Wb1)
    b1c = jnp.concatenate([ba1, bb1])
    W2chunk = (jnp.asarray(E4) * W1c[None, :, None, :]).reshape(128, 256)
    B2 = jnp.tile(b1c, 4).reshape(1, 256)

    W2c = _blockdiag2(Wa2, Wb2)
    b2c = jnp.concatenate([ba2, bb2])
    W3chunk = (jnp.asarray(E4) * W2c[None, :, None, :]).reshape(256, 128)
    B3 = jnp.tile(b2c, 4).reshape(1, 128)

    W3c = _blockdiag2(Wa3, Wb3)
    b3c = jnp.concatenate([ba3, bb3])

    nodes = np.arange(DIM)
    sub_perm = (nodes - 1) % DIM
    sup_perm = (nodes + 1) % DIM
    group_cols = [nodes, sub_perm, nodes, nodes, sub_perm, nodes, sup_perm]
    P = np.zeros((DIM, 1, 7, DIM), np.float32)
    for g in range(7):
        for n in range(DIM):
            P[n, 0, g, group_cols[g][n]] = 1.0
    W4 = (jnp.asarray(P) * W3c[None, :, :, None]).reshape(DIM * 32, 7 * DIM)
    B4 = jnp.repeat(b3c, DIM).reshape(1, 7 * DIM)
    return G1, B1, W2chunk, B2, W3chunk, B3, W4, B4


def kernel(x, Wa0, ba0, Wa1, ba1, Wa2, ba2, Wa3, ba3,
           Wb0, bb0, Wb1, bb1, Wb2, bb2, Wb3, bb3):
    batch = x.shape[0]
    G1, B1, W2chunk, B2, W3chunk, B3, W4, B4 = _build_weights(
        Wa0, ba0, Wa1, ba1, Wa2, ba2, Wa3, ba3,
        Wb0, bb0, Wb1, bb1, Wb2, bb2, Wb3, bb3)

    grid = (batch // BBLK,)
    full = lambda shape: pl.BlockSpec(shape, lambda b: (0,) * len(shape))
    f1, g1, f2, g2 = pl.pallas_call(
        _fwd_kernel,
        grid=grid,
        in_specs=[
            pl.BlockSpec((BBLK, DIM), lambda b: (b, 0)),
            full(G1.shape), full(B1.shape),
            full(W2chunk.shape), full(B2.shape),
            full(W3chunk.shape), full(B3.shape),
            full(W4.shape), full(B4.shape),
        ],
        out_specs=[
            pl.BlockSpec((BBLK, DIM), lambda b: (b, 0)),
            pl.BlockSpec((BBLK, DIM, DIM), lambda b: (b, 0, 0)),
            pl.BlockSpec((BBLK, DIM), lambda b: (b, 0)),
            pl.BlockSpec((BBLK, DIM, DIM), lambda b: (b, 0, 0)),
        ],
        out_shape=[
            jax.ShapeDtypeStruct((batch, DIM), jnp.float32),
            jax.ShapeDtypeStruct((batch, DIM, DIM), jnp.float32),
            jax.ShapeDtypeStruct((batch, DIM), jnp.float32),
            jax.ShapeDtypeStruct((batch, DIM, DIM), jnp.float32),
        ],
    )(x, G1, B1, W2chunk, B2, W3chunk, B3, W4, B4)
    return (f1.reshape(batch, DIM, 1), g1, f2.reshape(batch, DIM, 1), g2)


# speedup vs baseline: 3.8787x; 2.2640x over previous
import numpy as np
import jax
import jax.numpy as jnp
from jax.experimental import pallas as pl

DIM = 20
BBLK = 1024


def _dot(a, b, precision=jax.lax.Precision.DEFAULT):
    return jax.lax.dot_general(
        a, b, (((1,), (0,)), ((), ())),
        precision=precision,
        preferred_element_type=jnp.float32)


def _fwd_kernel(x_ref, g1w_ref, b1_ref, w2_ref, b2_ref, w3_ref, b3_ref,
                w4_ref, b4_ref, f1_ref, g1_ref, f2_ref, g2_ref):
    x = x_ref[...]
    h1 = jnp.maximum(
        _dot(x, g1w_ref[...], jax.lax.Precision.HIGHEST) + b1_ref[...],
        0.0)
    w2 = w2_ref[...]
    w3 = w3_ref[...]
    b2 = b2_ref[...]
    b3 = b3_ref[...]
    h3_parts = []
    for kt in range(5):
        h1k = h1[:, 128 * kt:128 * kt + 128]
        h2k = jnp.maximum(_dot(h1k, w2) + b2, 0.0)
        h3k = jnp.maximum(_dot(h2k, w3) + b3, 0.0)
        h3_parts.append(h3k)
    h3 = jnp.concatenate(h3_parts, axis=1)
    o = _dot(h3, w4_ref[...],
             jax.lax.Precision.HIGHEST) + b4_ref[...]
    ot = o.T

    f1_ref[...] = ot[0:20, :]
    f2_ref[...] = ot[60:80, :]

    ii = jax.lax.broadcasted_iota(jnp.int32, (DIM, DIM, 1), 0)
    jj = jax.lax.broadcasted_iota(jnp.int32, (DIM, DIM, 1), 1)
    dd = (jj - ii + DIM) % DIM

    sa = ot[20:40, :][None, :, :]
    da = ot[40:60, :][None, :, :]
    g1_ref[...] = jnp.where(dd == 0, da, jnp.where(dd == DIM - 1, sa, 0.0))

    sb = ot[80:100, :][None, :, :]
    db = ot[100:120, :][None, :, :]
    pb = ot[120:140, :][None, :, :]
    g2_ref[...] = jnp.where(
        dd == 0, db,
        jnp.where(dd == DIM - 1, sb, jnp.where(dd == 1, pb, 0.0)))


def _blockdiag2(A, B):
    za = jnp.zeros((A.shape[0], B.shape[1]), A.dtype)
    zb = jnp.zeros((B.shape[0], A.shape[1]), A.dtype)
    return jnp.concatenate(
        [jnp.concatenate([A, za], axis=1),
         jnp.concatenate([zb, B], axis=1)], axis=0)


def _build_weights(Wa0, ba0, Wa1, ba1, Wa2, ba2, Wa3, ba3,
                   Wb0, bb0, Wb1, bb1, Wb2, bb2, Wb3, bb3):
    f32 = jnp.float32
    W0c = jnp.concatenate(
        [Wa0, jnp.concatenate([jnp.zeros((1, 16), f32), Wb0], axis=0)],
        axis=1)
    b0c = jnp.concatenate([ba0, bb0])

    M = np.zeros((3, DIM, DIM, 1), np.float32)
    for t in range(3):
        for i in range(DIM):
            M[t, (i + t - 1) % DIM, i, 0] = 1.0
    G1 = (jnp.asarray(M) * W0c[:, None, None, :]).sum(0)
    G1 = G1.reshape(DIM, DIM * 32)
    B1 = jnp.tile(b0c, DIM).reshape(1, DIM * 32)

    E4 = np.eye(4, dtype=np.float32)[:, None, :, None]
    W1c = _blockdiag2(Wa1, Wb1)
    b1c = jnp.concatenate([ba1, bb1])
    W2chunk = (jnp.asarray(E4) * W1c[None, :, None, :]).reshape(128, 256)
    B2 = jnp.tile(b1c, 4).reshape(1, 256)

    W2c = _blockdiag2(Wa2, Wb2)
    b2c = jnp.concatenate([ba2, bb2])
    W3chunk = (jnp.asarray(E4) * W2c[None, :, None, :]).reshape(256, 128)
    B3 = jnp.tile(b2c, 4).reshape(1, 128)

    W3c = _blockdiag2(Wa3, Wb3)
    b3c = jnp.concatenate([ba3, bb3])

    nodes = np.arange(DIM)
    sub_perm = (nodes - 1) % DIM
    sup_perm = (nodes + 1) % DIM
    group_cols = [nodes, sub_perm, nodes, nodes, sub_perm, nodes, sup_perm]
    P = np.zeros((DIM, 1, 7, DIM), np.float32)
    for g in range(7):
        for n in range(DIM):
            P[n, 0, g, group_cols[g][n]] = 1.0
    W4 = (jnp.asarray(P) * W3c[None, :, :, None]).reshape(DIM * 32, 7 * DIM)
    B4 = jnp.repeat(b3c, DIM).reshape(1, 7 * DIM)
    return G1, B1, W2chunk, B2, W3chunk, B3, W4, B4


def kernel(x, Wa0, ba0, Wa1, ba1, Wa2, ba2, Wa3, ba3,
           Wb0, bb0, Wb1, bb1, Wb2, bb2, Wb3, bb3):
    batch = x.shape[0]
    G1, B1, W2chunk, B2, W3chunk, B3, W4, B4 = _build_weights(
        Wa0, ba0, Wa1, ba1, Wa2, ba2, Wa3, ba3,
        Wb0, bb0, Wb1, bb1, Wb2, bb2, Wb3, bb3)

    grid = (batch // BBLK,)
    full = lambda shape: pl.BlockSpec(shape, lambda b: (0,) * len(shape))
    f1t, g1t, f2t, g2t = pl.pallas_call(
        _fwd_kernel,
        grid=grid,
        in_specs=[
            pl.BlockSpec((BBLK, DIM), lambda b: (b, 0)),
            full(G1.shape), full(B1.shape),
            full(W2chunk.shape), full(B2.shape),
            full(W3chunk.shape), full(B3.shape),
            full(W4.shape), full(B4.shape),
        ],
        out_specs=[
            pl.BlockSpec((DIM, BBLK), lambda b: (0, b)),
            pl.BlockSpec((DIM, DIM, BBLK), lambda b: (0, 0, b)),
            pl.BlockSpec((DIM, BBLK), lambda b: (0, b)),
            pl.BlockSpec((DIM, DIM, BBLK), lambda b: (0, 0, b)),
        ],
        out_shape=[
            jax.ShapeDtypeStruct((DIM, batch), jnp.float32),
            jax.ShapeDtypeStruct((DIM, DIM, batch), jnp.float32),
            jax.ShapeDtypeStruct((DIM, batch), jnp.float32),
            jax.ShapeDtypeStruct((DIM, DIM, batch), jnp.float32),
        ],
    )(x, G1, B1, W2chunk, B2, W3chunk, B3, W4, B4)
    f1 = jnp.transpose(f1t, (1, 0))[:, :, None]
    f2 = jnp.transpose(f2t, (1, 0))[:, :, None]
    g1 = jnp.transpose(g1t, (2, 0, 1))
    g2 = jnp.transpose(g2t, (2, 0, 1))
    return (f1, g1, f2, g2)


# speedup vs baseline: 4.3233x; 1.1146x over previous
import numpy as np
import jax
import jax.numpy as jnp
from jax.experimental import pallas as pl

DIM = 20
BBLK = 1024


def _dotb(a, b):
    return jax.lax.dot_general(
        a, b, (((1,), (0,)), ((), ())),
        precision=jax.lax.Precision.DEFAULT,
        preferred_element_type=jnp.float32)


def _dot3(a, whi, wlo):
    ahi = a.astype(jnp.bfloat16)
    alo = (a - ahi.astype(jnp.float32)).astype(jnp.bfloat16)
    return _dotb(ahi, whi) + (_dotb(ahi, wlo) + _dotb(alo, whi))


_MD = np.zeros((DIM, DIM, 1), np.float32)
_MS = np.zeros((DIM, DIM, 1), np.float32)
_MP = np.zeros((DIM, DIM, 1), np.float32)
for _i in range(DIM):
    _MD[_i, _i, 0] = 1.0
    _MS[_i, (_i - 1) % DIM, 0] = 1.0
    _MP[_i, (_i + 1) % DIM, 0] = 1.0


def _fwd_kernel(x_ref, g1whi_ref, g1wlo_ref, b1_ref, w2hi_ref, w2lo_ref,
                b2_ref, w3hi_ref, w3lo_ref, b3_ref, w4hi_ref, w4lo_ref,
                b4_ref, md_ref, ms_ref, mp_ref, f1_ref, g1_ref, f2_ref,
                g2_ref):
    x = x_ref[...]
    h1 = jnp.maximum(
        _dot3(x, g1whi_ref[...], g1wlo_ref[...]) + b1_ref[...],
        0.0)
    w2hi = w2hi_ref[...]
    w2lo = w2lo_ref[...]
    w3hi = w3hi_ref[...]
    w3lo = w3lo_ref[...]
    b2 = b2_ref[...]
    b3 = b3_ref[...]
    h3_parts = []
    for kt in range(5):
        h1k = h1[:, 128 * kt:128 * kt + 128]
        h2k = jnp.maximum(_dot3(h1k, w2hi, w2lo) + b2, 0.0)
        h3k = jnp.maximum(_dot3(h2k, w3hi, w3lo) + b3, 0.0)
        h3_parts.append(h3k)
    h3 = jnp.concatenate(h3_parts, axis=1)
    o = _dot3(h3, w4hi_ref[...], w4lo_ref[...]) + b4_ref[...]
    ot = o.T

    f1_ref[...] = ot[0:20, :]
    f2_ref[...] = ot[60:80, :]

    md = md_ref[...]
    ms = ms_ref[...]
    mp = mp_ref[...]

    sa = ot[20:40, :][None, :, :]
    da = ot[40:60, :][None, :, :]
    g1_ref[...] = md * da + ms * sa

    sb = ot[80:100, :][None, :, :]
    db = ot[100:120, :][None, :, :]
    pb = ot[120:140, :][None, :, :]
    g2_ref[...] = md * db + (ms * sb + mp * pb)


def _blockdiag2(A, B):
    za = jnp.zeros((A.shape[0], B.shape[1]), A.dtype)
    zb = jnp.zeros((B.shape[0], A.shape[1]), A.dtype)
    return jnp.concatenate(
        [jnp.concatenate([A, za], axis=1),
         jnp.concatenate([zb, B], axis=1)], axis=0)


def _build_weights(Wa0, ba0, Wa1, ba1, Wa2, ba2, Wa3, ba3,
                   Wb0, bb0, Wb1, bb1, Wb2, bb2, Wb3, bb3):
    f32 = jnp.float32
    W0c = jnp.concatenate(
        [Wa0, jnp.concatenate([jnp.zeros((1, 16), f32), Wb0], axis=0)],
        axis=1)
    b0c = jnp.concatenate([ba0, bb0])

    M = np.zeros((3, DIM, DIM, 1), np.float32)
    for t in range(3):
        for i in range(DIM):
            M[t, (i + t - 1) % DIM, i, 0] = 1.0
    G1 = (jnp.asarray(M) * W0c[:, None, None, :]).sum(0)
    G1 = G1.reshape(DIM, DIM * 32)
    B1 = jnp.tile(b0c, DIM).reshape(1, DIM * 32)

    E4 = np.eye(4, dtype=np.float32)[:, None, :, None]
    W1c = _blockdiag2(Wa1, Wb1)
    b1c = jnp.concatenate([ba1, bb1])
    W2chunk = (jnp.asarray(E4) * W1c[None, :, None, :]).reshape(128, 256)
    B2 = jnp.tile(b1c, 4).reshape(1, 256)

    W2c = _blockdiag2(Wa2, Wb2)
    b2c = jnp.concatenate([ba2, bb2])
    W3chunk = (jnp.asarray(E4) * W2c[None, :, None, :]).reshape(256, 128)
    B3 = jnp.tile(b2c, 4).reshape(1, 128)

    W3c = _blockdiag2(Wa3, Wb3)
    b3c = jnp.concatenate([ba3, bb3])

    nodes = np.arange(DIM)
    sub_perm = (nodes - 1) % DIM
    sup_perm = (nodes + 1) % DIM
    group_cols = [nodes, sub_perm, nodes, nodes, sub_perm, nodes, sup_perm]
    P = np.zeros((DIM, 1, 7, DIM), np.float32)
    for g in range(7):
        for n in range(DIM):
            P[n, 0, g, group_cols[g][n]] = 1.0
    W4 = (jnp.asarray(P) * W3c[None, :, :, None]).reshape(DIM * 32, 7 * DIM)
    B4 = jnp.repeat(b3c, DIM).reshape(1, 7 * DIM)
    return G1, B1, W2chunk, B2, W3chunk, B3, W4, B4


def kernel(x, Wa0, ba0, Wa1, ba1, Wa2, ba2, Wa3, ba3,
           Wb0, bb0, Wb1, bb1, Wb2, bb2, Wb3, bb3):
    batch = x.shape[0]
    G1, B1, W2chunk, B2, W3chunk, B3, W4, B4 = _build_weights(
        Wa0, ba0, Wa1, ba1, Wa2, ba2, Wa3, ba3,
        Wb0, bb0, Wb1, bb1, Wb2, bb2, Wb3, bb3)

    def split(w):
        hi = w.astype(jnp.bfloat16)
        lo = (w - hi.astype(jnp.float32)).astype(jnp.bfloat16)
        return hi, lo

    G1hi, G1lo = split(G1)
    W2hi, W2lo = split(W2chunk)
    W3hi, W3lo = split(W3chunk)
    W4hi, W4lo = split(W4)

    grid = (batch // BBLK,)
    full = lambda shape: pl.BlockSpec(shape, lambda b: (0,) * len(shape))
    f1t, g1t, f2t, g2t = pl.pallas_call(
        _fwd_kernel,
        grid=grid,
        in_specs=[
            pl.BlockSpec((BBLK, DIM), lambda b: (b, 0)),
            full(G1.shape), full(G1.shape), full(B1.shape),
            full(W2chunk.shape), full(W2chunk.shape), full(B2.shape),
            full(W3chunk.shape), full(W3chunk.shape), full(B3.shape),
            full(W4.shape), full(W4.shape), full(B4.shape),
            full((DIM, DIM, 1)), full((DIM, DIM, 1)), full((DIM, DIM, 1)),
        ],
        out_specs=[
            pl.BlockSpec((DIM, BBLK), lambda b: (0, b)),
            pl.BlockSpec((DIM, DIM, BBLK), lambda b: (0, 0, b)),
            pl.BlockSpec((DIM, BBLK), lambda b: (0, b)),
            pl.BlockSpec((DIM, DIM, BBLK), lambda b: (0, 0, b)),
        ],
        out_shape=[
            jax.ShapeDtypeStruct((DIM, batch), jnp.float32),
            jax.ShapeDtypeStruct((DIM, DIM, batch), jnp.float32),
            jax.ShapeDtypeStruct((DIM, batch), jnp.float32),
            jax.ShapeDtypeStruct((DIM, DIM, batch), jnp.float32),
        ],
    )(x, G1hi, G1lo, B1, W2hi, W2lo, B2, W3hi, W3lo, B3, W4hi, W4lo, B4,
      jnp.asarray(_MD), jnp.asarray(_MS), jnp.asarray(_MP))
    f1 = jnp.transpose(f1t, (1, 0))[:, :, None]
    f2 = jnp.transpose(f2t, (1, 0))[:, :, None]
    g1 = jnp.transpose(g1t, (2, 0, 1))
    g2 = jnp.transpose(g2t, (2, 0, 1))
    return (f1, g1, f2, g2)


# speedup vs baseline: 9.3794x; 2.1695x over previous
import numpy as np
import jax
import jax.numpy as jnp
from jax.experimental import pallas as pl

DIM = 20
BBLK = 1024


def _dotb(a, b):
    return jax.lax.dot_general(
        a, b, (((1,), (0,)), ((), ())),
        precision=jax.lax.Precision.DEFAULT,
        preferred_element_type=jnp.float32)


_MD = np.zeros((DIM, DIM, 1), np.float32)
_MS = np.zeros((DIM, DIM, 1), np.float32)
_MP = np.zeros((DIM, DIM, 1), np.float32)
for _i in range(DIM):
    _MD[_i, _i, 0] = 1.0
    _MS[_i, (_i - 1) % DIM, 0] = 1.0
    _MP[_i, (_i + 1) % DIM, 0] = 1.0


def _fwd_kernel(x_ref, g1w_ref, b1_ref, w2_ref, b2_ref, w3_ref, b3_ref,
                w4_ref, b4_ref, md_ref, ms_ref, mp_ref, f1_ref, g1_ref,
                f2_ref, g2_ref):
    x = x_ref[...]
    h1 = jnp.maximum(_dotb(x, g1w_ref[...]) + b1_ref[...],
                     0.0)
    w2 = w2_ref[...]
    w3 = w3_ref[...]
    b2 = b2_ref[...]
    b3 = b3_ref[...]
    h3_parts = []
    for kt in range(5):
        h1k = h1[:, 128 * kt:128 * kt + 128]
        h2k = jnp.maximum(_dotb(h1k, w2) + b2, 0.0)
        h3k = jnp.maximum(_dotb(h2k, w3) + b3, 0.0)
        h3_parts.append(h3k)
    h3 = jnp.concatenate(h3_parts, axis=1)
    o = _dotb(h3, w4_ref[...]) + b4_ref[...]
    ot = o.T

    f1_ref[...] = ot[0:20, :]
    f2_ref[...] = ot[60:80, :]

    md = md_ref[...]
    ms = ms_ref[...]
    mp = mp_ref[...]

    sa = ot[20:40, :][None, :, :]
    da = ot[40:60, :][None, :, :]
    g1_ref[...] = md * da + ms * sa

    sb = ot[80:100, :][None, :, :]
    db = ot[100:120, :][None, :, :]
    pb = ot[120:140, :][None, :, :]
    g2_ref[...] = md * db + (ms * sb + mp * pb)


def _blockdiag2(A, B):
    za = jnp.zeros((A.shape[0], B.shape[1]), A.dtype)
    zb = jnp.zeros((B.shape[0], A.shape[1]), A.dtype)
    return jnp.concatenate(
        [jnp.concatenate([A, za], axis=1),
         jnp.concatenate([zb, B], axis=1)], axis=0)


def _build_weights(Wa0, ba0, Wa1, ba1, Wa2, ba2, Wa3, ba3,
                   Wb0, bb0, Wb1, bb1, Wb2, bb2, Wb3, bb3):
    f32 = jnp.float32
    W0c = jnp.concatenate(
        [Wa0, jnp.concatenate([jnp.zeros((1, 16), f32), Wb0], axis=0)],
        axis=1)
    b0c = jnp.concatenate([ba0, bb0])

    M = np.zeros((3, DIM, DIM, 1), np.float32)
    for t in range(3):
        for i in range(DIM):
            M[t, (i + t - 1) % DIM, i, 0] = 1.0
    G1 = (jnp.asarray(M) * W0c[:, None, None, :]).sum(0)
    G1 = G1.reshape(DIM, DIM * 32)
    B1 = jnp.tile(b0c, DIM).reshape(1, DIM * 32)

    E4 = np.eye(4, dtype=np.float32)[:, None, :, None]
    W1c = _blockdiag2(Wa1, Wb1)
    b1c = jnp.concatenate([ba1, bb1])
    W2chunk = (jnp.asarray(E4) * W1c[None, :, None, :]).reshape(128, 256)
    B2 = jnp.tile(b1c, 4).reshape(1, 256)

    W2c = _blockdiag2(Wa2, Wb2)
    b2c = jnp.concatenate([ba2, bb2])
    W3chunk = (jnp.asarray(E4) * W2c[None, :, None, :]).reshape(256, 128)
    B3 = jnp.tile(b2c, 4).reshape(1, 128)

    W3c = _blockdiag2(Wa3, Wb3)
    b3c = jnp.concatenate([ba3, bb3])

    nodes = np.arange(DIM)
    sub_perm = (nodes - 1) % DIM
    sup_perm = (nodes + 1) % DIM
    group_cols = [nodes, sub_perm, nodes, nodes, sub_perm, nodes, sup_perm]
    P = np.zeros((DIM, 1, 7, DIM), np.float32)
    for g in range(7):
        for n in range(DIM):
            P[n, 0, g, group_cols[g][n]] = 1.0
    W4 = (jnp.asarray(P) * W3c[None, :, :, None]).reshape(DIM * 32, 7 * DIM)
    B4 = jnp.repeat(b3c, DIM).reshape(1, 7 * DIM)
    return G1, B1, W2chunk, B2, W3chunk, B3, W4, B4


def kernel(x, Wa0, ba0, Wa1, ba1, Wa2, ba2, Wa3, ba3,
           Wb0, bb0, Wb1, bb1, Wb2, bb2, Wb3, bb3):
    batch = x.shape[0]
    G1, B1, W2chunk, B2, W3chunk, B3, W4, B4 = _build_weights(
        Wa0, ba0, Wa1, ba1, Wa2, ba2, Wa3, ba3,
        Wb0, bb0, Wb1, bb1, Wb2, bb2, Wb3, bb3)

    grid = (batch // BBLK,)
    full = lambda shape: pl.BlockSpec(shape, lambda b: (0,) * len(shape))
    f1t, g1t, f2t, g2t = pl.pallas_call(
        _fwd_kernel,
        grid=grid,
        in_specs=[
            pl.BlockSpec((BBLK, DIM), lambda b: (b, 0)),
            full(G1.shape), full(B1.shape),
            full(W2chunk.shape), full(B2.shape),
            full(W3chunk.shape), full(B3.shape),
            full(W4.shape), full(B4.shape),
            full((DIM, DIM, 1)), full((DIM, DIM, 1)), full((DIM, DIM, 1)),
        ],
        out_specs=[
            pl.BlockSpec((DIM, BBLK), lambda b: (0, b)),
            pl.BlockSpec((DIM, DIM, BBLK), lambda b: (0, 0, b)),
            pl.BlockSpec((DIM, BBLK), lambda b: (0, b)),
            pl.BlockSpec((DIM, DIM, BBLK), lambda b: (0, 0, b)),
        ],
        out_shape=[
            jax.ShapeDtypeStruct((DIM, batch), jnp.float32),
            jax.ShapeDtypeStruct((DIM, DIM, batch), jnp.float32),
            jax.ShapeDtypeStruct((DIM, batch), jnp.float32),
            jax.ShapeDtypeStruct((DIM, DIM, batch), jnp.float32),
        ],
    )(x, G1, B1, W2chunk, B2, W3chunk, B3, W4, B4,
      jnp.asarray(_MD), jnp.asarray(_MS), jnp.asarray(_MP))
    f1 = jnp.transpose(f1t, (1, 0))[:, :, None]
    f2 = jnp.transpose(f2t, (1, 0))[:, :, None]
    g1 = jnp.transpose(g1t, (2, 0, 1))
    g2 = jnp.transpose(g2t, (2, 0, 1))
    return (f1, g1, f2, g2)


# speedup vs baseline: 9.3959x; 1.0018x over previous
import numpy as np
import jax
import jax.numpy as jnp
from jax.experimental import pallas as pl

DIM = 20
BBLK = 2048


def _dotb(a, b):
    return jax.lax.dot_general(
        a, b, (((1,), (0,)), ((), ())),
        precision=jax.lax.Precision.DEFAULT,
        preferred_element_type=jnp.float32)


_MD = np.zeros((DIM, DIM, 1), np.float32)
_MS = np.zeros((DIM, DIM, 1), np.float32)
_MP = np.zeros((DIM, DIM, 1), np.float32)
for _i in range(DIM):
    _MD[_i, _i, 0] = 1.0
    _MS[_i, (_i - 1) % DIM, 0] = 1.0
    _MP[_i, (_i + 1) % DIM, 0] = 1.0


def _fwd_kernel(x_ref, g1w_ref, b1_ref, w2_ref, b2_ref, w3_ref, b3_ref,
                w4_ref, b4_ref, md_ref, ms_ref, mp_ref, f1_ref, g1_ref,
                f2_ref, g2_ref):
    x = x_ref[...]
    h1 = jnp.maximum(_dotb(x, g1w_ref[...]) + b1_ref[...],
                     0.0)
    w2 = w2_ref[...]
    w3 = w3_ref[...]
    b2 = b2_ref[...]
    b3 = b3_ref[...]
    h3_parts = []
    for kt in range(5):
        h1k = h1[:, 128 * kt:128 * kt + 128]
        h2k = jnp.maximum(_dotb(h1k, w2) + b2, 0.0)
        h3k = jnp.maximum(_dotb(h2k, w3) + b3, 0.0)
        h3_parts.append(h3k)
    h3 = jnp.concatenate(h3_parts, axis=1)
    o = _dotb(h3, w4_ref[...]) + b4_ref[...]
    ot = o.T

    f1_ref[...] = ot[0:20, :]
    f2_ref[...] = ot[60:80, :]

    md = md_ref[...]
    ms = ms_ref[...]
    mp = mp_ref[...]

    sa = ot[20:40, :][None, :, :]
    da = ot[40:60, :][None, :, :]
    g1_ref[...] = md * da + ms * sa

    sb = ot[80:100, :][None, :, :]
    db = ot[100:120, :][None, :, :]
    pb = ot[120:140, :][None, :, :]
    g2_ref[...] = md * db + (ms * sb + mp * pb)


def _blockdiag2(A, B):
    za = jnp.zeros((A.shape[0], B.shape[1]), A.dtype)
    zb = jnp.zeros((B.shape[0], A.shape[1]), A.dtype)
    return jnp.concatenate(
        [jnp.concatenate([A, za], axis=1),
         jnp.concatenate([zb, B], axis=1)], axis=0)


def _build_weights(Wa0, ba0, Wa1, ba1, Wa2, ba2, Wa3, ba3,
                   Wb0, bb0, Wb1, bb1, Wb2, bb2, Wb3, bb3):
    f32 = jnp.float32
    W0c = jnp.concatenate(
        [Wa0, jnp.concatenate([jnp.zeros((1, 16), f32), Wb0], axis=0)],
        axis=1)
    b0c = jnp.concatenate([ba0, bb0])

    M = np.zeros((3, DIM, DIM, 1), np.float32)
    for t in range(3):
        for i in range(DIM):
            M[t, (i + t - 1) % DIM, i, 0] = 1.0
    G1 = (jnp.asarray(M) * W0c[:, None, None, :]).sum(0)
    G1 = G1.reshape(DIM, DIM * 32)
    B1 = jnp.tile(b0c, DIM).reshape(1, DIM * 32)

    E4 = np.eye(4, dtype=np.float32)[:, None, :, None]
    W1c = _blockdiag2(Wa1, Wb1)
    b1c = jnp.concatenate([ba1, bb1])
    W2chunk = (jnp.asarray(E4) * W1c[None, :, None, :]).reshape(128, 256)
    B2 = jnp.tile(b1c, 4).reshape(1, 256)

    W2c = _blockdiag2(Wa2, Wb2)
    b2c = jnp.concatenate([ba2, bb2])
    W3chunk = (jnp.asarray(E4) * W2c[None, :, None, :]).reshape(256, 128)
    B3 = jnp.tile(b2c, 4).reshape(1, 128)

    W3c = _blockdiag2(Wa3, Wb3)
    b3c = jnp.concatenate([ba3, bb3])

    nodes = np.arange(DIM)
    sub_perm = (nodes - 1) % DIM
    sup_perm = (nodes + 1) % DIM
    group_cols = [nodes, sub_perm, nodes, nodes, sub_perm, nodes, sup_perm]
    P = np.zeros((DIM, 1, 7, DIM), np.float32)
    for g in range(7):
        for n in range(DIM):
            P[n, 0, g, group_cols[g][n]] = 1.0
    W4 = (jnp.asarray(P) * W3c[None, :, :, None]).reshape(DIM * 32, 7 * DIM)
    B4 = jnp.repeat(b3c, DIM).reshape(1, 7 * DIM)
    return G1, B1, W2chunk, B2, W3chunk, B3, W4, B4


def kernel(x, Wa0, ba0, Wa1, ba1, Wa2, ba2, Wa3, ba3,
           Wb0, bb0, Wb1, bb1, Wb2, bb2, Wb3, bb3):
    batch = x.shape[0]
    G1, B1, W2chunk, B2, W3chunk, B3, W4, B4 = _build_weights(
        Wa0, ba0, Wa1, ba1, Wa2, ba2, Wa3, ba3,
        Wb0, bb0, Wb1, bb1, Wb2, bb2, Wb3, bb3)

    grid = (batch // BBLK,)
    full = lambda shape: pl.BlockSpec(shape, lambda b: (0,) * len(shape))
    f1t, g1t, f2t, g2t = pl.pallas_call(
        _fwd_kernel,
        grid=grid,
        in_specs=[
            pl.BlockSpec((BBLK, DIM), lambda b: (b, 0)),
            full(G1.shape), full(B1.shape),
            full(W2chunk.shape), full(B2.shape),
            full(W3chunk.shape), full(B3.shape),
            full(W4.shape), full(B4.shape),
            full((DIM, DIM, 1)), full((DIM, DIM, 1)), full((DIM, DIM, 1)),
        ],
        out_specs=[
            pl.BlockSpec((DIM, BBLK), lambda b: (0, b)),
            pl.BlockSpec((DIM, DIM, BBLK), lambda b: (0, 0, b)),
            pl.BlockSpec((DIM, BBLK), lambda b: (0, b)),
            pl.BlockSpec((DIM, DIM, BBLK), lambda b: (0, 0, b)),
        ],
        out_shape=[
            jax.ShapeDtypeStruct((DIM, batch), jnp.float32),
            jax.ShapeDtypeStruct((DIM, DIM, batch), jnp.float32),
            jax.ShapeDtypeStruct((DIM, batch), jnp.float32),
            jax.ShapeDtypeStruct((DIM, DIM, batch), jnp.float32),
        ],
    )(x, G1, B1, W2chunk, B2, W3chunk, B3, W4, B4,
      jnp.asarray(_MD), jnp.asarray(_MS), jnp.asarray(_MP))
    f1 = jnp.transpose(f1t, (1, 0))[:, :, None]
    f2 = jnp.transpose(f2t, (1, 0))[:, :, None]
    g1 = jnp.transpose(g1t, (2, 0, 1))
    g2 = jnp.transpose(g2t, (2, 0, 1))
    return (f1, g1, f2, g2)


# speedup vs baseline: 10.2070x; 1.0863x over previous
import numpy as np
import jax
import jax.numpy as jnp
from jax.experimental import pallas as pl

DIM = 20
BBLK = 2048


def _dotb(a, b):
    return jax.lax.dot_general(
        a, b, (((1,), (0,)), ((), ())),
        precision=jax.lax.Precision.DEFAULT,
        preferred_element_type=jnp.float32)


_MD = np.zeros((DIM, DIM, 1), np.float32)
_MS = np.zeros((DIM, DIM, 1), np.float32)
_MP = np.zeros((DIM, DIM, 1), np.float32)
for _i in range(DIM):
    _MD[_i, _i, 0] = 1.0
    _MS[_i, (_i - 1) % DIM, 0] = 1.0
    _MP[_i, (_i + 1) % DIM, 0] = 1.0


def _fwd_kernel(x_ref, g1w_ref, b1_ref, w2_ref, b2_ref, w3_ref, b3_ref,
                w4_ref, b4_ref, md_ref, ms_ref, mp_ref, f1_ref, g1_ref,
                f2_ref, g2_ref):
    x = x_ref[...]
    h1 = jnp.maximum(_dotb(x, g1w_ref[...]) + b1_ref[...],
                     0.0)
    w2 = w2_ref[...]
    w3 = w3_ref[...]
    b2 = b2_ref[...]
    b3 = b3_ref[...]
    h3_parts = []
    for kt in range(5):
        h1k = h1[:, 128 * kt:128 * kt + 128]
        h2k = jnp.maximum(_dotb(h1k, w2) + b2, 0.0)
        h3k = jnp.maximum(_dotb(h2k, w3) + b3, 0.0)
        h3_parts.append(h3k)
    h3 = jnp.concatenate(h3_parts, axis=1)
    o = _dotb(h3, w4_ref[...]) + b4_ref[...]
    ot = o.T

    f1_ref[...] = ot[0:20, :]
    f2_ref[...] = ot[60:80, :]

    md = md_ref[...]
    ms = ms_ref[...]
    mp = mp_ref[...]

    sa = ot[20:40, :][None, :, :]
    da = ot[40:60, :][None, :, :]
    g1_ref[...] = md * da + ms * sa

    sb = ot[80:100, :][None, :, :]
    db = ot[100:120, :][None, :, :]
    pb = ot[120:140, :][None, :, :]
    g2_ref[...] = md * db + (ms * sb + mp * pb)


def _blockdiag2(A, B):
    za = jnp.zeros((A.shape[0], B.shape[1]), A.dtype)
    zb = jnp.zeros((B.shape[0], A.shape[1]), A.dtype)
    return jnp.concatenate(
        [jnp.concatenate([A, za], axis=1),
         jnp.concatenate([zb, B], axis=1)], axis=0)


_M2 = np.zeros((3, DIM, DIM * 32), np.float32)
for _t in range(3):
    for _i in range(DIM):
        _M2[_t, (_i + _t - 1) % DIM, 32 * _i:32 * _i + 32] = 1.0
_E2 = np.kron(np.eye(4, dtype=np.float32), np.ones((32, 64), np.float32))
_E3 = np.kron(np.eye(4, dtype=np.float32), np.ones((64, 32), np.float32))
_P2 = np.zeros((DIM * 32, 7 * DIM), np.float32)
_sub = lambda n: (n - 1) % DIM
_sup = lambda n: (n + 1) % DIM
_gcols = [lambda n: n, _sub, lambda n: n, lambda n: n, _sub, lambda n: n, _sup]
for _g in range(7):
    for _n in range(DIM):
        _P2[32 * _n:32 * _n + 32, DIM * _g + _gcols[_g](_n)] = 1.0


def _build_kernel(Wa0_ref, ba0_ref, Wa1_ref, ba1_ref, Wa2_ref, ba2_ref,
                  Wa3_ref, ba3_ref, Wb0_ref, bb0_ref, Wb1_ref, bb1_ref,
                  Wb2_ref, bb2_ref, Wb3_ref, bb3_ref, m2_ref, e2_ref,
                  e3_ref, p2_ref, g1w_ref, b1_ref, w2_ref, b2_ref, w3_ref,
                  b3_ref, w4_ref, b4_ref):
    cat = jnp.concatenate
    W0c = cat([Wa0_ref[...],
               cat([jnp.zeros((1, 16), jnp.float32), Wb0_ref[...]], axis=0)],
              axis=1)
    W0r = cat([W0c] * DIM, axis=1)
    g1w_ref[...] = (m2_ref[...] * W0r[:, None, :]).sum(0)
    b0c = cat([ba0_ref[...], bb0_ref[...]], axis=1)
    b1_ref[...] = cat([b0c] * DIM, axis=1)

    W1c = _blockdiag2(Wa1_ref[...], Wb1_ref[...])
    W1r = cat([cat([W1c] * 4, axis=1)] * 4, axis=0)
    w2_ref[...] = e2_ref[...] * W1r
    b1c = cat([ba1_ref[...], bb1_ref[...]], axis=1)
    b2_ref[...] = cat([b1c] * 4, axis=1)

    W2c = _blockdiag2(Wa2_ref[...], Wb2_ref[...])
    W2r = cat([cat([W2c] * 4, axis=1)] * 4, axis=0)
    w3_ref[...] = e3_ref[...] * W2r
    b2c = cat([ba2_ref[...], bb2_ref[...]], axis=1)
    b3_ref[...] = cat([b2c] * 4, axis=1)

    W3c = _blockdiag2(Wa3_ref[...], Wb3_ref[...])
    W3g = cat([jnp.broadcast_to(W3c[:, g:g + 1], (32, DIM))
               for g in range(7)], axis=1)
    W3r = cat([W3g] * DIM, axis=0)
    w4_ref[...] = p2_ref[...] * W3r
    b3c = cat([ba3_ref[...], bb3_ref[...]], axis=1)
    b4_ref[...] = cat([jnp.broadcast_to(b3c[:, g:g + 1], (1, DIM))
                       for g in range(7)], axis=1)


def _build_weights(Wa0, ba0, Wa1, ba1, Wa2, ba2, Wa3, ba3,
                   Wb0, bb0, Wb1, bb1, Wb2, bb2, Wb3, bb3):
    f32 = jnp.float32
    W0c = jnp.concatenate(
        [Wa0, jnp.concatenate([jnp.zeros((1, 16), f32), Wb0], axis=0)],
        axis=1)
    b0c = jnp.concatenate([ba0, bb0])

    M = np.zeros((3, DIM, DIM, 1), np.float32)
    for t in range(3):
        for i in range(DIM):
            M[t, (i + t - 1) % DIM, i, 0] = 1.0
    G1 = (jnp.asarray(M) * W0c[:, None, None, :]).sum(0)
    G1 = G1.reshape(DIM, DIM * 32)
    B1 = jnp.tile(b0c, DIM).reshape(1, DIM * 32)

    E4 = np.eye(4, dtype=np.float32)[:, None, :, None]
    W1c = _blockdiag2(Wa1, Wb1)
    b1c = jnp.concatenate([ba1, bb1])
    W2chunk = (jnp.asarray(E4) * W1c[None, :, None, :]).reshape(128, 256)
    B2 = jnp.tile(b1c, 4).reshape(1, 256)

    W2c = _blockdiag2(Wa2, Wb2)
    b2c = jnp.concatenate([ba2, bb2])
    W3chunk = (jnp.asarray(E4) * W2c[None, :, None, :]).reshape(256, 128)
    B3 = jnp.tile(b2c, 4).reshape(1, 128)

    W3c = _blockdiag2(Wa3, Wb3)
    b3c = jnp.concatenate([ba3, bb3])

    nodes = np.arange(DIM)
    sub_perm = (nodes - 1) % DIM
    sup_perm = (nodes + 1) % DIM
    group_cols = [nodes, sub_perm, nodes, nodes, sub_perm, nodes, sup_perm]
    P = np.zeros((DIM, 1, 7, DIM), np.float32)
    for g in range(7):
        for n in range(DIM):
            P[n, 0, g, group_cols[g][n]] = 1.0
    W4 = (jnp.asarray(P) * W3c[None, :, :, None]).reshape(DIM * 32, 7 * DIM)
    B4 = jnp.repeat(b3c, DIM).reshape(1, 7 * DIM)
    return G1, B1, W2chunk, B2, W3chunk, B3, W4, B4


def kernel(x, Wa0, ba0, Wa1, ba1, Wa2, ba2, Wa3, ba3,
           Wb0, bb0, Wb1, bb1, Wb2, bb2, Wb3, bb3):
    batch = x.shape[0]
    full = lambda shape: pl.BlockSpec(shape, lambda b: (0,) * len(shape))
    fullw = lambda shape: pl.BlockSpec(shape, lambda: (0,) * len(shape))

    raw = [Wa0, ba0.reshape(1, -1), Wa1, ba1.reshape(1, -1),
           Wa2, ba2.reshape(1, -1), Wa3, ba3.reshape(1, -1),
           Wb0, bb0.reshape(1, -1), Wb1, bb1.reshape(1, -1),
           Wb2, bb2.reshape(1, -1), Wb3, bb3.reshape(1, -1),
           jnp.asarray(_M2), jnp.asarray(_E2), jnp.asarray(_E3),
           jnp.asarray(_P2)]
    G1, B1, W2chunk, B2, W3chunk, B3, W4, B4 = pl.pallas_call(
        _build_kernel,
        in_specs=[fullw(a.shape) for a in raw],
        out_specs=[
            fullw((DIM, DIM * 32)), fullw((1, DIM * 32)),
            fullw((128, 256)), fullw((1, 256)),
            fullw((256, 128)), fullw((1, 128)),
            fullw((DIM * 32, 7 * DIM)), fullw((1, 7 * DIM)),
        ],
        out_shape=[
            jax.ShapeDtypeStruct((DIM, DIM * 32), jnp.float32),
            jax.ShapeDtypeStruct((1, DIM * 32), jnp.float32),
            jax.ShapeDtypeStruct((128, 256), jnp.float32),
            jax.ShapeDtypeStruct((1, 256), jnp.float32),
            jax.ShapeDtypeStruct((256, 128), jnp.float32),
            jax.ShapeDtypeStruct((1, 128), jnp.float32),
            jax.ShapeDtypeStruct((DIM * 32, 7 * DIM), jnp.float32),
            jax.ShapeDtypeStruct((1, 7 * DIM), jnp.float32),
        ],
    )(*raw)

    grid = (batch // BBLK,)
    f1t, g1t, f2t, g2t = pl.pallas_call(
        _fwd_kernel,
        grid=grid,
        in_specs=[
            pl.BlockSpec((BBLK, DIM), lambda b: (b, 0)),
            full(G1.shape), full(B1.shape),
            full(W2chunk.shape), full(B2.shape),
            full(W3chunk.shape), full(B3.shape),
            full(W4.shape), full(B4.shape),
            full((DIM, DIM, 1)), full((DIM, DIM, 1)), full((DIM, DIM, 1)),
        ],
        out_specs=[
            pl.BlockSpec((DIM, BBLK), lambda b: (0, b)),
            pl.BlockSpec((DIM, DIM, BBLK), lambda b: (0, 0, b)),
            pl.BlockSpec((DIM, BBLK), lambda b: (0, b)),
            pl.BlockSpec((DIM, DIM, BBLK), lambda b: (0, 0, b)),
        ],
        out_shape=[
            jax.ShapeDtypeStruct((DIM, batch), jnp.float32),
            jax.ShapeDtypeStruct((DIM, DIM, batch), jnp.float32),
            jax.ShapeDtypeStruct((DIM, batch), jnp.float32),
            jax.ShapeDtypeStruct((DIM, DIM, batch), jnp.float32),
        ],
    )(x, G1, B1, W2chunk, B2, W3chunk, B3, W4, B4,
      jnp.asarray(_MD), jnp.asarray(_MS), jnp.asarray(_MP))
    f1 = jnp.transpose(f1t, (1, 0))[:, :, None]
    f2 = jnp.transpose(f2t, (1, 0))[:, :, None]
    g1 = jnp.transpose(g1t, (2, 0, 1))
    g2 = jnp.transpose(g2t, (2, 0, 1))
    return (f1, g1, f2, g2)


# speedup vs baseline: 11.1439x; 1.0918x over previous
import numpy as np
import jax
import jax.numpy as jnp
from jax.experimental import pallas as pl

DIM = 20
BBLK = 2048


def _dotb(a, b):
    return jax.lax.dot_general(
        a, b, (((1,), (0,)), ((), ())),
        precision=jax.lax.Precision.DEFAULT,
        preferred_element_type=jnp.float32)


_MD = np.zeros((DIM, DIM, 1), np.float32)
_MS = np.zeros((DIM, DIM, 1), np.float32)
_MP = np.zeros((DIM, DIM, 1), np.float32)
for _i in range(DIM):
    _MD[_i, _i, 0] = 1.0
    _MS[_i, (_i - 1) % DIM, 0] = 1.0
    _MP[_i, (_i + 1) % DIM, 0] = 1.0


def _fwd_kernel(x_ref, g1w_ref, b1_ref, w2_ref, b2_ref, w3_ref, b3_ref,
                w4_ref, b4_ref, md_ref, ms_ref, mp_ref, f1_ref, g1_ref,
                f2_ref, g2_ref):
    x = x_ref[...]
    h1 = jnp.maximum(_dotb(x, g1w_ref[...]) + b1_ref[...],
                     0.0)
    w2 = w2_ref[...]
    w3 = w3_ref[...]
    b2 = b2_ref[...]
    b3 = b3_ref[...]
    h3_parts = []
    for kt in range(5):
        h1k = h1[:, 128 * kt:128 * kt + 128]
        h2k = jnp.maximum(_dotb(h1k, w2) + b2, 0.0)
        h3k = jnp.maximum(_dotb(h2k, w3) + b3, 0.0)
        h3_parts.append(h3k)
    h3 = jnp.concatenate(h3_parts, axis=1)
    o = _dotb(h3, w4_ref[...]) + b4_ref[...]
    ot = o.T

    f1_ref[...] = ot[0:20, :][:, None, :]
    f2_ref[...] = ot[60:80, :][:, None, :]

    md = md_ref[...]
    ms = ms_ref[...]
    mp = mp_ref[...]

    sa = ot[20:40, :][None, :, :]
    da = ot[40:60, :][None, :, :]
    g1_ref[...] = md * da + ms * sa

    sb = ot[80:100, :][None, :, :]
    db = ot[100:120, :][None, :, :]
    pb = ot[120:140, :][None, :, :]
    g2_ref[...] = md * db + (ms * sb + mp * pb)


def _blockdiag2(A, B):
    za = jnp.zeros((A.shape[0], B.shape[1]), A.dtype)
    zb = jnp.zeros((B.shape[0], A.shape[1]), A.dtype)
    return jnp.concatenate(
        [jnp.concatenate([A, za], axis=1),
         jnp.concatenate([zb, B], axis=1)], axis=0)


_M2 = np.zeros((3, DIM, DIM * 32), np.float32)
for _t in range(3):
    for _i in range(DIM):
        _M2[_t, (_i + _t - 1) % DIM, 32 * _i:32 * _i + 32] = 1.0
_E2 = np.kron(np.eye(4, dtype=np.float32), np.ones((32, 64), np.float32))
_E3 = np.kron(np.eye(4, dtype=np.float32), np.ones((64, 32), np.float32))
_P2 = np.zeros((DIM * 32, 7 * DIM), np.float32)
_sub = lambda n: (n - 1) % DIM
_sup = lambda n: (n + 1) % DIM
_gcols = [lambda n: n, _sub, lambda n: n, lambda n: n, _sub, lambda n: n, _sup]
for _g in range(7):
    for _n in range(DIM):
        _P2[32 * _n:32 * _n + 32, DIM * _g + _gcols[_g](_n)] = 1.0


def _build_kernel(Wa0_ref, ba0_ref, Wa1_ref, ba1_ref, Wa2_ref, ba2_ref,
                  Wa3_ref, ba3_ref, Wb0_ref, bb0_ref, Wb1_ref, bb1_ref,
                  Wb2_ref, bb2_ref, Wb3_ref, bb3_ref, m2_ref, e2_ref,
                  e3_ref, p2_ref, g1w_ref, b1_ref, w2_ref, b2_ref, w3_ref,
                  b3_ref, w4_ref, b4_ref):
    cat = jnp.concatenate
    W0c = cat([Wa0_ref[...],
               cat([jnp.zeros((1, 16), jnp.float32), Wb0_ref[...]], axis=0)],
              axis=1)
    W0r = cat([W0c] * DIM, axis=1)
    g1w_ref[...] = (m2_ref[...] * W0r[:, None, :]).sum(0)
    b0c = cat([ba0_ref[...], bb0_ref[...]], axis=1)
    b1_ref[...] = cat([b0c] * DIM, axis=1)

    W1c = _blockdiag2(Wa1_ref[...], Wb1_ref[...])
    W1r = cat([cat([W1c] * 4, axis=1)] * 4, axis=0)
    w2_ref[...] = e2_ref[...] * W1r
    b1c = cat([ba1_ref[...], bb1_ref[...]], axis=1)
    b2_ref[...] = cat([b1c] * 4, axis=1)

    W2c = _blockdiag2(Wa2_ref[...], Wb2_ref[...])
    W2r = cat([cat([W2c] * 4, axis=1)] * 4, axis=0)
    w3_ref[...] = e3_ref[...] * W2r
    b2c = cat([ba2_ref[...], bb2_ref[...]], axis=1)
    b3_ref[...] = cat([b2c] * 4, axis=1)

    W3c = _blockdiag2(Wa3_ref[...], Wb3_ref[...])
    W3g = cat([jnp.broadcast_to(W3c[:, g:g + 1], (32, DIM))
               for g in range(7)], axis=1)
    W3r = cat([W3g] * DIM, axis=0)
    w4_ref[...] = p2_ref[...] * W3r
    b3c = cat([ba3_ref[...], bb3_ref[...]], axis=1)
    b4_ref[...] = cat([jnp.broadcast_to(b3c[:, g:g + 1], (1, DIM))
                       for g in range(7)], axis=1)


def _build_weights(Wa0, ba0, Wa1, ba1, Wa2, ba2, Wa3, ba3,
                   Wb0, bb0, Wb1, bb1, Wb2, bb2, Wb3, bb3):
    f32 = jnp.float32
    W0c = jnp.concatenate(
        [Wa0, jnp.concatenate([jnp.zeros((1, 16), f32), Wb0], axis=0)],
        axis=1)
    b0c = jnp.concatenate([ba0, bb0])

    M = np.zeros((3, DIM, DIM, 1), np.float32)
    for t in range(3):
        for i in range(DIM):
            M[t, (i + t - 1) % DIM, i, 0] = 1.0
    G1 = (jnp.asarray(M) * W0c[:, None, None, :]).sum(0)
    G1 = G1.reshape(DIM, DIM * 32)
    B1 = jnp.tile(b0c, DIM).reshape(1, DIM * 32)

    E4 = np.eye(4, dtype=np.float32)[:, None, :, None]
    W1c = _blockdiag2(Wa1, Wb1)
    b1c = jnp.concatenate([ba1, bb1])
    W2chunk = (jnp.asarray(E4) * W1c[None, :, None, :]).reshape(128, 256)
    B2 = jnp.tile(b1c, 4).reshape(1, 256)

    W2c = _blockdiag2(Wa2, Wb2)
    b2c = jnp.concatenate([ba2, bb2])
    W3chunk = (jnp.asarray(E4) * W2c[None, :, None, :]).reshape(256, 128)
    B3 = jnp.tile(b2c, 4).reshape(1, 128)

    W3c = _blockdiag2(Wa3, Wb3)
    b3c = jnp.concatenate([ba3, bb3])

    nodes = np.arange(DIM)
    sub_perm = (nodes - 1) % DIM
    sup_perm = (nodes + 1) % DIM
    group_cols = [nodes, sub_perm, nodes, nodes, sub_perm, nodes, sup_perm]
    P = np.zeros((DIM, 1, 7, DIM), np.float32)
    for g in range(7):
        for n in range(DIM):
            P[n, 0, g, group_cols[g][n]] = 1.0
    W4 = (jnp.asarray(P) * W3c[None, :, :, None]).reshape(DIM * 32, 7 * DIM)
    B4 = jnp.repeat(b3c, DIM).reshape(1, 7 * DIM)
    return G1, B1, W2chunk, B2, W3chunk, B3, W4, B4


def kernel(x, Wa0, ba0, Wa1, ba1, Wa2, ba2, Wa3, ba3,
           Wb0, bb0, Wb1, bb1, Wb2, bb2, Wb3, bb3):
    batch = x.shape[0]
    full = lambda shape: pl.BlockSpec(shape, lambda b: (0,) * len(shape))
    fullw = lambda shape: pl.BlockSpec(shape, lambda: (0,) * len(shape))

    raw = [Wa0, ba0.reshape(1, -1), Wa1, ba1.reshape(1, -1),
           Wa2, ba2.reshape(1, -1), Wa3, ba3.reshape(1, -1),
           Wb0, bb0.reshape(1, -1), Wb1, bb1.reshape(1, -1),
           Wb2, bb2.reshape(1, -1), Wb3, bb3.reshape(1, -1),
           jnp.asarray(_M2), jnp.asarray(_E2), jnp.asarray(_E3),
           jnp.asarray(_P2)]
    G1, B1, W2chunk, B2, W3chunk, B3, W4, B4 = pl.pallas_call(
        _build_kernel,
        in_specs=[fullw(a.shape) for a in raw],
        out_specs=[
            fullw((DIM, DIM * 32)), fullw((1, DIM * 32)),
            fullw((128, 256)), fullw((1, 256)),
            fullw((256, 128)), fullw((1, 128)),
            fullw((DIM * 32, 7 * DIM)), fullw((1, 7 * DIM)),
        ],
        out_shape=[
            jax.ShapeDtypeStruct((DIM, DIM * 32), jnp.float32),
            jax.ShapeDtypeStruct((1, DIM * 32), jnp.float32),
            jax.ShapeDtypeStruct((128, 256), jnp.float32),
            jax.ShapeDtypeStruct((1, 256), jnp.float32),
            jax.ShapeDtypeStruct((256, 128), jnp.float32),
            jax.ShapeDtypeStruct((1, 128), jnp.float32),
            jax.ShapeDtypeStruct((DIM * 32, 7 * DIM), jnp.float32),
            jax.ShapeDtypeStruct((1, 7 * DIM), jnp.float32),
        ],
    )(*raw)

    grid = (batch // BBLK,)
    f1t, g1t, f2t, g2t = pl.pallas_call(
        _fwd_kernel,
        grid=grid,
        in_specs=[
            pl.BlockSpec((BBLK, DIM), lambda b: (b, 0)),
            full(G1.shape), full(B1.shape),
            full(W2chunk.shape), full(B2.shape),
            full(W3chunk.shape), full(B3.shape),
            full(W4.shape), full(B4.shape),
            full((DIM, DIM, 1)), full((DIM, DIM, 1)), full((DIM, DIM, 1)),
        ],
        out_specs=[
            pl.BlockSpec((DIM, 1, BBLK), lambda b: (0, 0, b)),
            pl.BlockSpec((DIM, DIM, BBLK), lambda b: (0, 0, b)),
            pl.BlockSpec((DIM, 1, BBLK), lambda b: (0, 0, b)),
            pl.BlockSpec((DIM, DIM, BBLK), lambda b: (0, 0, b)),
        ],
        out_shape=[
            jax.ShapeDtypeStruct((DIM, 1, batch), jnp.float32),
            jax.ShapeDtypeStruct((DIM, DIM, batch), jnp.float32),
            jax.ShapeDtypeStruct((DIM, 1, batch), jnp.float32),
            jax.ShapeDtypeStruct((DIM, DIM, batch), jnp.float32),
        ],
    )(x, G1, B1, W2chunk, B2, W3chunk, B3, W4, B4,
      jnp.asarray(_MD), jnp.asarray(_MS), jnp.asarray(_MP))
    f1 = jnp.transpose(f1t, (2, 0, 1))
    f2 = jnp.transpose(f2t, (2, 0, 1))
    g1 = jnp.transpose(g1t, (2, 0, 1))
    g2 = jnp.transpose(g2t, (2, 0, 1))
    return (f1, g1, f2, g2)
